# baseline (speedup 1.0000x reference)
import functools

import jax
import jax.numpy as jnp
from jax import lax
from jax.experimental import pallas as pl
from jax.experimental.pallas import tpu as pltpu
from jax.experimental.pallas import tpu_sc as plsc

_NC = 2
_NS = 16
_NW = _NC * _NS
_C = 128


def _make_sc_segment_sum(num_node, d, n_chunks):
    k_per_tile = n_chunks // _NW
    zrows = ((num_node + _NS - 1) // _NS + 7) // 8 * 8
    n_pad = _NS * zrows

    mesh = plsc.VectorSubcoreMesh(core_axis_name="c", subcore_axis_name="s",
                                  num_cores=_NC)

    @functools.partial(
        pl.kernel,
        out_type=jax.ShapeDtypeStruct((_NC * n_pad, d), jnp.float32),
        mesh=mesh,
        scratch_types=[
            pltpu.VMEM((2, _C), jnp.int32),
            pltpu.VMEM((_C, d), jnp.float32),
            pltpu.VMEM((8, d), jnp.float32),
            pltpu.VMEM_SHARED((n_pad, d), jnp.float32),
            pltpu.SemaphoreType.DMA,
        ],
    )
    def sc_fn(x_hbm, idx_hbm, part_hbm, idx_v, rows_v, zbuf, acc, sem):
        core = lax.axis_index("c")
        sub = lax.axis_index("s")
        wid = sub * _NC + core

        def zstore(i, carry):
            r = i // (d // 16)
            cc = (i % (d // 16)) * 16
            zbuf[r, pl.ds(cc, 16)] = jnp.zeros((16,), jnp.float32)
            return carry

        lax.fori_loop(0, 8 * (d // 16), zstore, 0)

        def zcopy(k, carry):
            pltpu.sync_copy(zbuf, acc.at[pl.ds(sub * zrows + k * 8, 8)])
            return carry

        lax.fori_loop(0, zrows // 8, zcopy, 0)
        plsc.subcore_barrier()

        def chunk_body(i, carry):
            c = wid * k_per_tile + i
            pltpu.sync_copy(idx_hbm.at[c], idx_v)
            pltpu.async_copy(x_hbm.at[idx_v.at[0]], rows_v, sem).wait()
            pltpu.sync_copy(rows_v, acc.at[idx_v.at[1]], add=True)
            return carry

        lax.fori_loop(0, k_per_tile, chunk_body, 0)
        plsc.subcore_barrier()

        pltpu.sync_copy(acc.at[pl.ds(sub * zrows, zrows)],
                        part_hbm.at[pl.ds(core * n_pad + sub * zrows, zrows)])

    return sc_fn, n_pad


def _mm_body(p0_ref, p1_ref, w_ref, o_ref):
    p = p0_ref[...] + p1_ref[...]
    o_ref[...] = lax.dot_general(p, w_ref[...], (((1,), (1,)), ((), ())),
                                 preferred_element_type=jnp.float32)


def kernel(x, edge_index, edge_attr, W_fc, W_edge, W_att):
    num_node, d = x.shape
    num_edge = edge_index.shape[1]

    row = edge_index[0].astype(jnp.int32)
    col = edge_index[1].astype(jnp.int32)

    n_chunks = -(-num_edge // (_C * 2 * _NW)) * 2 * _NW
    e_pad = n_chunks * _C
    col = jnp.pad(col, (0, e_pad - num_edge)).reshape(n_chunks, 1, _C)
    row = jnp.pad(row, (0, e_pad - num_edge),
                  constant_values=num_node).reshape(n_chunks, 1, _C)
    idxcat = jnp.concatenate([col, row], axis=1)

    sc_fn, n_pad = _make_sc_segment_sum(num_node, d, n_chunks)
    part = sc_fn(x, idxcat)

    blk = n_pad // _NS
    grid = n_pad // blk
    z_pad = pl.pallas_call(
        _mm_body,
        grid=(grid,),
        in_specs=[
            pl.BlockSpec((blk, d), lambda i: (i, 0)),
            pl.BlockSpec((blk, d), lambda i: (i + grid, 0)),
            pl.BlockSpec((d, d), lambda i: (0, 0)),
        ],
        out_specs=pl.BlockSpec((blk, d), lambda i: (i, 0)),
        out_shape=jax.ShapeDtypeStruct((n_pad, d), jnp.float32),
    )(part, part, W_fc)
    return z_pad[:num_node]


# speedup vs baseline: 1.1776x; 1.1776x over previous
import functools

import jax
import jax.numpy as jnp
from jax import lax
from jax.experimental import pallas as pl
from jax.experimental.pallas import tpu as pltpu
from jax.experimental.pallas import tpu_sc as plsc

_NC = 2
_NS = 16
_NW = _NC * _NS
_C = 128


def _make_sc_segment_sum(num_node, d, n_chunks):
    k_per_tile = n_chunks // _NW
    zrows = ((num_node + _NS - 1) // _NS + 7) // 8 * 8
    n_pad = _NS * zrows

    mesh = plsc.VectorSubcoreMesh(core_axis_name="c", subcore_axis_name="s",
                                  num_cores=_NC)

    G = 4
    n_groups = k_per_tile // G

    @functools.partial(
        pl.kernel,
        out_type=jax.ShapeDtypeStruct((_NC * n_pad, d), jnp.float32),
        mesh=mesh,
        scratch_types=[
            pltpu.VMEM((2, G, 2, _C), jnp.int32),
            pltpu.VMEM((2, _C, d), jnp.float32),
            pltpu.VMEM((8, d), jnp.float32),
            pltpu.VMEM_SHARED((n_pad, d), jnp.float32),
            pltpu.SemaphoreType.DMA,
            pltpu.SemaphoreType.DMA,
        ],
    )
    def sc_fn(x_hbm, idx_hbm, part_hbm, idxg, rows2, zbuf, acc, sem0, sem1):
        core = lax.axis_index("c")
        sub = lax.axis_index("s")
        wid = sub * _NC + core
        base = wid * k_per_tile
        sems = (sem0, sem1)

        def zstore(i, carry):
            r = i // (d // 16)
            cc = (i % (d // 16)) * 16
            zbuf[r, pl.ds(cc, 16)] = jnp.zeros((16,), jnp.float32)
            return carry

        lax.fori_loop(0, 8 * (d // 16), zstore, 0)

        def zcopy(k, carry):
            pltpu.sync_copy(zbuf, acc.at[pl.ds(sub * zrows + k * 8, 8)])
            return carry

        lax.fori_loop(0, zrows // 8, zcopy, 0)

        pltpu.sync_copy(idx_hbm.at[pl.ds(base, G)], idxg.at[0])
        pltpu.async_copy(x_hbm.at[idxg.at[0, 0, 0]], rows2.at[0], sems[0])
        plsc.subcore_barrier()

        def body(gg, carry):
            for gpar in range(2):
                g = gg * 2 + gpar
                gbase = base + g * G

                @pl.when(g < n_groups - 1)
                def _fetch():
                    pltpu.sync_copy(idx_hbm.at[pl.ds(gbase + G, G)],
                                    idxg.at[1 - gpar])

                for j in range(G):
                    s = j % 2
                    if j < G - 1:
                        pltpu.async_copy(x_hbm.at[idxg.at[gpar, j + 1, 0]],
                                         rows2.at[1 - s], sems[1 - s])
                    else:
                        @pl.when(g < n_groups - 1)
                        def _start():
                            pltpu.async_copy(x_hbm.at[idxg.at[1 - gpar, 0, 0]],
                                             rows2.at[1 - s], sems[1 - s])
                    pltpu.make_async_copy(x_hbm.at[idxg.at[gpar, j, 0]],
                                          rows2.at[s], sems[s]).wait()
                    pltpu.sync_copy(rows2.at[s], acc.at[idxg.at[gpar, j, 1]],
                                    add=True)
            return carry

        lax.fori_loop(0, n_groups // 2, body, 0)
        plsc.subcore_barrier()

        pltpu.sync_copy(acc.at[pl.ds(sub * zrows, zrows)],
                        part_hbm.at[pl.ds(core * n_pad + sub * zrows, zrows)])

    return sc_fn, n_pad


def _mm_body(p0_ref, p1_ref, w_ref, o_ref):
    p = p0_ref[...] + p1_ref[...]
    o_ref[...] = lax.dot_general(p, w_ref[...], (((1,), (1,)), ((), ())),
                                 preferred_element_type=jnp.float32)


def kernel(x, edge_index, edge_attr, W_fc, W_edge, W_att):
    num_node, d = x.shape
    num_edge = edge_index.shape[1]

    row = edge_index[0].astype(jnp.int32)
    col = edge_index[1].astype(jnp.int32)

    n_chunks = -(-num_edge // (_C * 8 * _NW)) * 8 * _NW
    e_pad = n_chunks * _C
    col = jnp.pad(col, (0, e_pad - num_edge)).reshape(n_chunks, 1, _C)
    row = jnp.pad(row, (0, e_pad - num_edge),
                  constant_values=num_node).reshape(n_chunks, 1, _C)
    idxcat = jnp.concatenate([col, row], axis=1)

    sc_fn, n_pad = _make_sc_segment_sum(num_node, d, n_chunks)
    part = sc_fn(x, idxcat)

    blk = n_pad // _NS
    grid = n_pad // blk
    z_pad = pl.pallas_call(
        _mm_body,
        grid=(grid,),
        in_specs=[
            pl.BlockSpec((blk, d), lambda i: (i, 0)),
            pl.BlockSpec((blk, d), lambda i: (i + grid, 0)),
            pl.BlockSpec((d, d), lambda i: (0, 0)),
        ],
        out_specs=pl.BlockSpec((blk, d), lambda i: (i, 0)),
        out_shape=jax.ShapeDtypeStruct((n_pad, d), jnp.float32),
    )(part, part, W_fc)
    return z_pad[:num_node]
